# baseline (speedup 1.0000x reference)
import functools

import jax
import jax.numpy as jnp
from jax import lax
from jax.experimental import pallas as pl
from jax.experimental.pallas import tpu as pltpu
from jax.experimental.pallas import tpu_sc as plsc

N = 2048
C = 768
E = 8
K = 2
H = 2048
HS = 2048
AUXC = 0.01

BLK = 256
NK = N * K
MAXB = NK // BLK + E
PMAX = MAXB * BLK
HBLK = 2048
NH = H // HBLK

SC_CORES = 2
SC_SUBCORES = 16
NW = SC_CORES * SC_SUBCORES
GCH = PMAX // NW // 2
TW = N // NW


def _router_body(xf_ref, rw_ref, idx_ref, w_ref, aux_ref):
    x = xf_ref[...]
    rw = rw_ref[...]
    logits = lax.dot_general(rw, x, (((1,), (1,)), ((), ())),
                             preferred_element_type=jnp.float32)
    e_iota = lax.broadcasted_iota(jnp.int32, (E, N), 0)
    m1 = jnp.max(logits, axis=0, keepdims=True)
    i1 = jnp.min(jnp.where(logits == m1, e_iota, E), axis=0, keepdims=True)
    masked = jnp.where(e_iota == i1, -jnp.inf, logits)
    m2 = jnp.max(masked, axis=0, keepdims=True)
    i2 = jnp.min(jnp.where(masked == m2, e_iota, E), axis=0, keepdims=True)
    t = jnp.exp(m2 - m1)
    w1 = 1.0 / (1.0 + t)
    w2 = t / (1.0 + t)
    z = jnp.exp(logits - m1)
    probs = z / jnp.sum(z, axis=0, keepdims=True)
    mean_probs = jnp.sum(probs, axis=1, keepdims=True) / N
    sel = (e_iota == i1).astype(jnp.float32) + (e_iota == i2).astype(jnp.float32)
    frac = jnp.sum(sel, axis=1, keepdims=True) / N
    aux_ref[...] = (AUXC * jnp.sum(frac * mean_probs)).reshape(1, 1)
    idx_ref[...] = jnp.concatenate([i1, i2], axis=0)
    w_ref[...] = jnp.concatenate([w1, w2], axis=0)


def _router(xf, router_w):
    return pl.pallas_call(
        _router_body,
        out_shape=(
            jax.ShapeDtypeStruct((K, N), jnp.int32),
            jax.ShapeDtypeStruct((K, N), jnp.float32),
            jax.ShapeDtypeStruct((1, 1), jnp.float32),
        ),
    )(xf, router_w)


def _shared_body(x_ref, sg_ref, su_ref, sd_ref, out_ref, sgb, sub, sdb):
    @pl.when(pl.program_id(0) == 0)
    def _():
        sgb[...] = sg_ref[...].astype(jnp.bfloat16)
        sub[...] = su_ref[...].astype(jnp.bfloat16)
        sdb[...] = sd_ref[...].astype(jnp.bfloat16)

    x = x_ref[...].astype(jnp.bfloat16)
    g = jnp.dot(x, sgb[...], preferred_element_type=jnp.float32)
    u = jnp.dot(x, sub[...], preferred_element_type=jnp.float32)
    h = (g * jax.nn.sigmoid(g) * u).astype(jnp.bfloat16)
    out_ref[...] = jnp.dot(h, sdb[...], preferred_element_type=jnp.float32)


def _shared(xf, sg, su, sd):
    TB = 512
    return pl.pallas_call(
        _shared_body,
        grid=(N // TB,),
        in_specs=[
            pl.BlockSpec((TB, C), lambda i: (i, 0)),
            pl.BlockSpec((C, HS), lambda i: (0, 0)),
            pl.BlockSpec((C, HS), lambda i: (0, 0)),
            pl.BlockSpec((HS, C), lambda i: (0, 0)),
        ],
        out_specs=pl.BlockSpec((TB, C), lambda i: (i, 0)),
        out_shape=jax.ShapeDtypeStruct((N, C), jnp.float32),
        scratch_shapes=[
            pltpu.VMEM((C, HS), jnp.bfloat16),
            pltpu.VMEM((C, HS), jnp.bfloat16),
            pltpu.VMEM((HS, C), jnp.bfloat16),
        ],
        compiler_params=pltpu.CompilerParams(
            dimension_semantics=("arbitrary",)),
    )(xf, sg, su, sd)


def _ffn_body(be_ref, xs_ref, eg_ref, eu_ref, ed_ref, ys_ref, egb, eub, edb):
    i = pl.program_id(0)
    cur = be_ref[i]
    prev = be_ref[jnp.maximum(i - 1, 0)]
    new_w = jnp.logical_or(i == 0, cur != prev)

    @pl.when(jnp.logical_and(cur < E, new_w))
    def _():
        egb[...] = eg_ref[0].astype(jnp.bfloat16)
        eub[...] = eu_ref[0].astype(jnp.bfloat16)
        edb[...] = ed_ref[0].astype(jnp.bfloat16)

    @pl.when(cur < E)
    def _():
        x = xs_ref[...].astype(jnp.bfloat16)
        g = jnp.dot(x, egb[...], preferred_element_type=jnp.float32)
        u = jnp.dot(x, eub[...], preferred_element_type=jnp.float32)
        h = (g * jax.nn.sigmoid(g) * u).astype(jnp.bfloat16)
        ys_ref[...] = jnp.dot(h, edb[...], preferred_element_type=jnp.float32)


def _ffn(block_expert, xs, eg, eu, ed):
    def emap(i, be):
        return (jnp.minimum(be[i], E - 1), 0, 0)

    def edmap(i, be):
        return (jnp.minimum(be[i], E - 1), 0, 0)

    grid_spec = pltpu.PrefetchScalarGridSpec(
        num_scalar_prefetch=1,
        grid=(MAXB,),
        in_specs=[
            pl.BlockSpec((BLK, C), lambda i, be: (i, 0)),
            pl.BlockSpec((1, C, H), emap),
            pl.BlockSpec((1, C, H), emap),
            pl.BlockSpec((1, H, C), edmap),
        ],
        out_specs=pl.BlockSpec((BLK, C), lambda i, be: (i, 0)),
        scratch_shapes=[
            pltpu.VMEM((C, H), jnp.bfloat16),
            pltpu.VMEM((C, H), jnp.bfloat16),
            pltpu.VMEM((H, C), jnp.bfloat16),
        ],
    )
    return pl.pallas_call(
        _ffn_body,
        grid_spec=grid_spec,
        out_shape=jax.ShapeDtypeStruct((PMAX, C), jnp.float32),
        compiler_params=pltpu.CompilerParams(
            dimension_semantics=("arbitrary",)),
    )(block_expert, xs, eg, eu, ed)


AW = NK // NW
HC = AW // 2


def _sc_scatter_dispatch(xf, sidx):
    mesh = plsc.VectorSubcoreMesh(core_axis_name="c", subcore_axis_name="s")

    @functools.partial(
        pl.kernel,
        mesh=mesh,
        out_type=jax.ShapeDtypeStruct((PMAX, C), jnp.float32),
        scratch_types=[
            pltpu.VMEM((2, HC), jnp.int32),
            pltpu.VMEM((HC, C), jnp.float32),
            pltpu.VMEM((HC, C), jnp.float32),
            pltpu.SemaphoreType.DMA,
            pltpu.SemaphoreType.DMA,
            pltpu.SemaphoreType.DMA,
            pltpu.SemaphoreType.DMA,
        ],
    )
    def k(xf_hbm, sidx_hbm, xs_hbm, idx_v, b0, b1, s0, s1, w0, w1):
        wid = lax.axis_index("s") * SC_CORES + lax.axis_index("c")
        base = (wid * AW) % N
        pltpu.sync_copy(sidx_hbm.at[wid], idx_v)
        r0 = pltpu.async_copy(xf_hbm.at[pl.ds(base, HC)], b0, s0)
        r1 = pltpu.async_copy(xf_hbm.at[pl.ds(base + HC, HC)], b1, s1)
        r0.wait()
        c0 = pltpu.async_copy(b0, xs_hbm.at[idx_v.at[0]], w0)
        r1.wait()
        c1 = pltpu.async_copy(b1, xs_hbm.at[idx_v.at[1]], w1)
        c0.wait()
        c1.wait()

    return k(xf, sidx)


def _sc_gather_out(ys, pos):
    mesh = plsc.VectorSubcoreMesh(core_axis_name="c", subcore_axis_name="s")

    @functools.partial(
        pl.kernel,
        mesh=mesh,
        out_type=jax.ShapeDtypeStruct((K * N, C), jnp.float32),
        scratch_types=[
            pltpu.VMEM((K, TW), jnp.int32),
            pltpu.VMEM((TW, C), jnp.float32),
            pltpu.VMEM((TW, C), jnp.float32),
            pltpu.SemaphoreType.DMA,
            pltpu.SemaphoreType.DMA,
            pltpu.SemaphoreType.DMA,
            pltpu.SemaphoreType.DMA,
        ],
    )
    def k(ys_hbm, pos_hbm, yg_hbm, idx_v, buf0, buf1, s0, s1, w0, w1):
        wid = lax.axis_index("s") * SC_CORES + lax.axis_index("c")
        pltpu.sync_copy(pos_hbm.at[wid], idx_v)
        g0 = pltpu.async_copy(ys_hbm.at[idx_v.at[0]], buf0, s0)
        g1 = pltpu.async_copy(ys_hbm.at[idx_v.at[1]], buf1, s1)
        g0.wait()
        c0 = pltpu.async_copy(buf0, yg_hbm.at[pl.ds(wid * TW, TW)], w0)
        g1.wait()
        c1 = pltpu.async_copy(buf1, yg_hbm.at[pl.ds(N + wid * TW, TW)], w1)
        c0.wait()
        c1.wait()

    return k(ys, pos)


def _final_body(sh_ref, y1_ref, y2_ref, w1_ref, w2_ref, out_ref):
    out_ref[...] = (sh_ref[...] + y1_ref[...] * w1_ref[...]
                    + y2_ref[...] * w2_ref[...])


def _final_add(shared, yg, w1, w2):
    TB = 512
    return pl.pallas_call(
        _final_body,
        grid=(N // TB,),
        in_specs=[
            pl.BlockSpec((TB, C), lambda i: (i, 0)),
            pl.BlockSpec((TB, C), lambda i: (i, 0)),
            pl.BlockSpec((TB, C), lambda i: (i + N // TB, 0)),
            pl.BlockSpec((TB, 1), lambda i: (i, 0)),
            pl.BlockSpec((TB, 1), lambda i: (i, 0)),
        ],
        out_specs=pl.BlockSpec((TB, C), lambda i: (i, 0)),
        out_shape=jax.ShapeDtypeStruct((N, C), jnp.float32),
        compiler_params=pltpu.CompilerParams(
            dimension_semantics=("arbitrary",)),
    )(shared, yg, yg, w1, w2)


def _dispatch_meta(idx_en):
    expert_flat = idx_en.reshape(-1)
    onehot = (expert_flat[:, None]
              == jnp.arange(E, dtype=expert_flat.dtype)[None, :]
              ).astype(jnp.int32)
    cum = jnp.cumsum(onehot, axis=0)
    counts = cum[-1]
    pcounts = ((counts + BLK - 1) // BLK) * BLK
    pend = jnp.cumsum(pcounts)
    pstarts = pend - pcounts
    ppos = jnp.sum(onehot * (cum + pstarts[None, :] - 1),
                   axis=1).astype(jnp.int32)
    block_expert = jnp.sum(
        pend[None, :] <= (jnp.arange(MAXB, dtype=jnp.int32) * BLK)[:, None],
        axis=1).astype(jnp.int32)
    sidx = ppos.reshape(NW, 2, HC)
    pos = ppos.reshape(K, NW, TW).transpose(1, 0, 2)
    return sidx, pos, block_expert


def kernel(x, router_w, eg, eu, ed, sg, su, sd):
    xf = x.reshape(N, C)
    idx_en, w_en, aux = _router(xf, router_w)
    sidx, pos, block_expert = _dispatch_meta(idx_en)
    xs = _sc_scatter_dispatch(xf, sidx)
    shared = _shared(xf, sg, su, sd)
    ys = _ffn(block_expert, xs, eg, eu, ed)
    yg = _sc_gather_out(ys, pos)
    w1 = w_en[0].reshape(N, 1)
    w2 = w_en[1].reshape(N, 1)
    final = _final_add(shared, yg, w1, w2)
    return final.reshape(x.shape), aux[0, 0]


# speedup vs baseline: 1.0436x; 1.0436x over previous
import functools

import jax
import jax.numpy as jnp
from jax import lax
from jax.experimental import pallas as pl
from jax.experimental.pallas import tpu as pltpu
from jax.experimental.pallas import tpu_sc as plsc

N = 2048
C = 768
E = 8
K = 2
H = 2048
HS = 2048
AUXC = 0.01

BLK = 256
NK = N * K
MAXB = NK // BLK + E
PMAX = MAXB * BLK
HBLK = 2048
NH = H // HBLK

SC_CORES = 2
SC_SUBCORES = 16
NW = SC_CORES * SC_SUBCORES
GCH = PMAX // NW // 2
TW = N // NW


def _router_body(xf_ref, rw_ref, idx_ref, w_ref, aux_ref):
    x = xf_ref[...]
    rw = rw_ref[...]
    logits = lax.dot_general(rw, x, (((1,), (1,)), ((), ())),
                             preferred_element_type=jnp.float32)
    e_iota = lax.broadcasted_iota(jnp.int32, (E, N), 0)
    m1 = jnp.max(logits, axis=0, keepdims=True)
    i1 = jnp.min(jnp.where(logits == m1, e_iota, E), axis=0, keepdims=True)
    masked = jnp.where(e_iota == i1, -jnp.inf, logits)
    m2 = jnp.max(masked, axis=0, keepdims=True)
    i2 = jnp.min(jnp.where(masked == m2, e_iota, E), axis=0, keepdims=True)
    t = jnp.exp(m2 - m1)
    w1 = 1.0 / (1.0 + t)
    w2 = t / (1.0 + t)
    z = jnp.exp(logits - m1)
    probs = z / jnp.sum(z, axis=0, keepdims=True)
    mean_probs = jnp.sum(probs, axis=1, keepdims=True) / N
    sel = (e_iota == i1).astype(jnp.float32) + (e_iota == i2).astype(jnp.float32)
    frac = jnp.sum(sel, axis=1, keepdims=True) / N
    aux_ref[...] = (AUXC * jnp.sum(frac * mean_probs)).reshape(1, 1)
    idx_ref[...] = jnp.concatenate([i1, i2], axis=0)
    w_ref[...] = jnp.concatenate([w1, w2], axis=0)


def _router(xf, router_w):
    return pl.pallas_call(
        _router_body,
        out_shape=(
            jax.ShapeDtypeStruct((K, N), jnp.int32),
            jax.ShapeDtypeStruct((K, N), jnp.float32),
            jax.ShapeDtypeStruct((1, 1), jnp.float32),
        ),
    )(xf, router_w)


def _shared_body(x_ref, sg_ref, su_ref, sd_ref, out_ref):
    x = x_ref[...]
    g = jnp.dot(x, sg_ref[...], preferred_element_type=jnp.float32)
    u = jnp.dot(x, su_ref[...], preferred_element_type=jnp.float32)
    h = g * jax.nn.sigmoid(g) * u
    out_ref[...] = jnp.dot(h, sd_ref[...], preferred_element_type=jnp.float32)


def _shared(xf, sg, su, sd):
    TB = 512
    return pl.pallas_call(
        _shared_body,
        grid=(N // TB,),
        in_specs=[
            pl.BlockSpec((TB, C), lambda i: (i, 0)),
            pl.BlockSpec((C, HS), lambda i: (0, 0)),
            pl.BlockSpec((C, HS), lambda i: (0, 0)),
            pl.BlockSpec((HS, C), lambda i: (0, 0)),
        ],
        out_specs=pl.BlockSpec((TB, C), lambda i: (i, 0)),
        out_shape=jax.ShapeDtypeStruct((N, C), jnp.float32),
        compiler_params=pltpu.CompilerParams(
            dimension_semantics=("arbitrary",)),
    )(xf, sg, su, sd)


def _ffn_body(be_ref, xs_ref, eg_ref, eu_ref, ed_ref, ys_ref):
    i = pl.program_id(0)

    @pl.when(be_ref[i] < E)
    def _():
        x = xs_ref[...]
        g = jnp.dot(x, eg_ref[0], preferred_element_type=jnp.float32)
        u = jnp.dot(x, eu_ref[0], preferred_element_type=jnp.float32)
        h = g * jax.nn.sigmoid(g) * u
        ys_ref[...] = jnp.dot(h, ed_ref[0], preferred_element_type=jnp.float32)


def _ffn(block_expert, xs, eg, eu, ed):
    def emap(i, be):
        return (jnp.minimum(be[i], E - 1), 0, 0)

    def edmap(i, be):
        return (jnp.minimum(be[i], E - 1), 0, 0)

    grid_spec = pltpu.PrefetchScalarGridSpec(
        num_scalar_prefetch=1,
        grid=(MAXB,),
        in_specs=[
            pl.BlockSpec((BLK, C), lambda i, be: (i, 0)),
            pl.BlockSpec((1, C, H), emap),
            pl.BlockSpec((1, C, H), emap),
            pl.BlockSpec((1, H, C), edmap),
        ],
        out_specs=pl.BlockSpec((BLK, C), lambda i, be: (i, 0)),
    )
    return pl.pallas_call(
        _ffn_body,
        grid_spec=grid_spec,
        out_shape=jax.ShapeDtypeStruct((PMAX, C), jnp.float32),
        compiler_params=pltpu.CompilerParams(
            dimension_semantics=("arbitrary",)),
    )(block_expert, xs, eg, eu, ed)


AW = NK // NW
HC = AW // 2


def _sc_scatter_dispatch(xf, sidx):
    mesh = plsc.VectorSubcoreMesh(core_axis_name="c", subcore_axis_name="s")

    @functools.partial(
        pl.kernel,
        mesh=mesh,
        out_type=jax.ShapeDtypeStruct((PMAX, C), jnp.float32),
        scratch_types=[
            pltpu.VMEM((2, HC), jnp.int32),
            pltpu.VMEM((HC, C), jnp.float32),
            pltpu.VMEM((HC, C), jnp.float32),
            pltpu.SemaphoreType.DMA,
            pltpu.SemaphoreType.DMA,
            pltpu.SemaphoreType.DMA,
            pltpu.SemaphoreType.DMA,
        ],
    )
    def k(xf_hbm, sidx_hbm, xs_hbm, idx_v, b0, b1, s0, s1, w0, w1):
        wid = lax.axis_index("s") * SC_CORES + lax.axis_index("c")
        base = (wid * AW) % N
        pltpu.sync_copy(sidx_hbm.at[wid], idx_v)
        r0 = pltpu.async_copy(xf_hbm.at[pl.ds(base, HC)], b0, s0)
        r1 = pltpu.async_copy(xf_hbm.at[pl.ds(base + HC, HC)], b1, s1)
        r0.wait()
        c0 = pltpu.async_copy(b0, xs_hbm.at[idx_v.at[0]], w0)
        r1.wait()
        c1 = pltpu.async_copy(b1, xs_hbm.at[idx_v.at[1]], w1)
        c0.wait()
        c1.wait()

    return k(xf, sidx)


def _sc_gather_out(ys, pos):
    mesh = plsc.VectorSubcoreMesh(core_axis_name="c", subcore_axis_name="s")

    @functools.partial(
        pl.kernel,
        mesh=mesh,
        out_type=jax.ShapeDtypeStruct((K * N, C), jnp.float32),
        scratch_types=[
            pltpu.VMEM((K, TW), jnp.int32),
            pltpu.VMEM((TW, C), jnp.float32),
            pltpu.VMEM((TW, C), jnp.float32),
            pltpu.SemaphoreType.DMA,
            pltpu.SemaphoreType.DMA,
            pltpu.SemaphoreType.DMA,
            pltpu.SemaphoreType.DMA,
        ],
    )
    def k(ys_hbm, pos_hbm, yg_hbm, idx_v, buf0, buf1, s0, s1, w0, w1):
        wid = lax.axis_index("s") * SC_CORES + lax.axis_index("c")
        pltpu.sync_copy(pos_hbm.at[wid], idx_v)
        g0 = pltpu.async_copy(ys_hbm.at[idx_v.at[0]], buf0, s0)
        g1 = pltpu.async_copy(ys_hbm.at[idx_v.at[1]], buf1, s1)
        g0.wait()
        c0 = pltpu.async_copy(buf0, yg_hbm.at[pl.ds(wid * TW, TW)], w0)
        g1.wait()
        c1 = pltpu.async_copy(buf1, yg_hbm.at[pl.ds(N + wid * TW, TW)], w1)
        c0.wait()
        c1.wait()

    return k(ys, pos)


def _final_body(sh_ref, y1_ref, y2_ref, w1_ref, w2_ref, out_ref):
    out_ref[...] = (sh_ref[...] + y1_ref[...] * w1_ref[...]
                    + y2_ref[...] * w2_ref[...])


def _final_add(shared, yg, w1, w2):
    TB = 512
    return pl.pallas_call(
        _final_body,
        grid=(N // TB,),
        in_specs=[
            pl.BlockSpec((TB, C), lambda i: (i, 0)),
            pl.BlockSpec((TB, C), lambda i: (i, 0)),
            pl.BlockSpec((TB, C), lambda i: (i + N // TB, 0)),
            pl.BlockSpec((TB, 1), lambda i: (i, 0)),
            pl.BlockSpec((TB, 1), lambda i: (i, 0)),
        ],
        out_specs=pl.BlockSpec((TB, C), lambda i: (i, 0)),
        out_shape=jax.ShapeDtypeStruct((N, C), jnp.float32),
        compiler_params=pltpu.CompilerParams(
            dimension_semantics=("arbitrary",)),
    )(shared, yg, yg, w1, w2)


def _dispatch_meta(idx_en):
    expert_flat = idx_en.reshape(-1)
    onehot = (expert_flat[:, None]
              == jnp.arange(E, dtype=expert_flat.dtype)[None, :]
              ).astype(jnp.int32)
    cum = jnp.cumsum(onehot, axis=0)
    counts = cum[-1]
    pcounts = ((counts + BLK - 1) // BLK) * BLK
    pend = jnp.cumsum(pcounts)
    pstarts = pend - pcounts
    ppos = jnp.sum(onehot * (cum + pstarts[None, :] - 1),
                   axis=1).astype(jnp.int32)
    block_expert = jnp.sum(
        pend[None, :] <= (jnp.arange(MAXB, dtype=jnp.int32) * BLK)[:, None],
        axis=1).astype(jnp.int32)
    sidx = ppos.reshape(NW, 2, HC)
    pos = ppos.reshape(K, NW, TW).transpose(1, 0, 2)
    return sidx, pos, block_expert


def kernel(x, router_w, eg, eu, ed, sg, su, sd):
    xf = x.reshape(N, C)
    idx_en, w_en, aux = _router(xf, router_w)
    sidx, pos, block_expert = _dispatch_meta(idx_en)
    xf_sh, _ = lax.optimization_barrier((xf, sidx))
    xs = _sc_scatter_dispatch(xf, sidx)
    shared = _shared(xf_sh, sg, su, sd)
    ys = _ffn(block_expert, xs, eg, eu, ed)
    yg = _sc_gather_out(ys, pos)
    w1 = w_en[0].reshape(N, 1)
    w2 = w_en[1].reshape(N, 1)
    final = _final_add(shared, yg, w1, w2)
    return final.reshape(x.shape), aux[0, 0]


# speedup vs baseline: 1.0702x; 1.0255x over previous
import functools

import jax
import jax.numpy as jnp
from jax import lax
from jax.experimental import pallas as pl
from jax.experimental.pallas import tpu as pltpu
from jax.experimental.pallas import tpu_sc as plsc

N = 2048
C = 768
E = 8
K = 2
H = 2048
HS = 2048
AUXC = 0.01

BLK = 256
NK = N * K
MAXB = NK // BLK + E
PMAX = MAXB * BLK

SC_CORES = 2
SC_SUBCORES = 16
NW = SC_CORES * SC_SUBCORES
TW = N // NW


def _router_body(xf_ref, rw_ref, idx_ref, w_ref, aux_ref):
    x = xf_ref[...]
    rw = rw_ref[...]
    logits = lax.dot_general(rw, x, (((1,), (1,)), ((), ())),
                             preferred_element_type=jnp.float32)
    e_iota = lax.broadcasted_iota(jnp.int32, (E, N), 0)
    m1 = jnp.max(logits, axis=0, keepdims=True)
    i1 = jnp.min(jnp.where(logits == m1, e_iota, E), axis=0, keepdims=True)
    masked = jnp.where(e_iota == i1, -jnp.inf, logits)
    m2 = jnp.max(masked, axis=0, keepdims=True)
    i2 = jnp.min(jnp.where(masked == m2, e_iota, E), axis=0, keepdims=True)
    t = jnp.exp(m2 - m1)
    w1 = 1.0 / (1.0 + t)
    w2 = t / (1.0 + t)
    z = jnp.exp(logits - m1)
    probs = z / jnp.sum(z, axis=0, keepdims=True)
    mean_probs = jnp.sum(probs, axis=1, keepdims=True) / N
    sel = (e_iota == i1).astype(jnp.float32) + (e_iota == i2).astype(jnp.float32)
    frac = jnp.sum(sel, axis=1, keepdims=True) / N
    aux_ref[...] = (AUXC * jnp.sum(frac * mean_probs)).reshape(1, 1)
    idx_ref[...] = jnp.concatenate([i1, i2], axis=0)
    w_ref[...] = jnp.concatenate([w1, w2], axis=0)


def _router(xf, router_w):
    return pl.pallas_call(
        _router_body,
        out_shape=(
            jax.ShapeDtypeStruct((K, N), jnp.int32),
            jax.ShapeDtypeStruct((K, N), jnp.float32),
            jax.ShapeDtypeStruct((1, 1), jnp.float32),
        ),
    )(xf, router_w)


def _shared_body(x_ref, sg_ref, su_ref, sd_ref, out_ref):
    x = x_ref[...]
    g = jnp.dot(x, sg_ref[...], preferred_element_type=jnp.float32)
    u = jnp.dot(x, su_ref[...], preferred_element_type=jnp.float32)
    h = g * jax.nn.sigmoid(g) * u
    out_ref[...] = jnp.dot(h, sd_ref[...], preferred_element_type=jnp.float32)


def _shared(xf, sg, su, sd):
    TB = 512
    return pl.pallas_call(
        _shared_body,
        grid=(N // TB,),
        in_specs=[
            pl.BlockSpec((TB, C), lambda i: (i, 0)),
            pl.BlockSpec((C, HS), lambda i: (0, 0)),
            pl.BlockSpec((C, HS), lambda i: (0, 0)),
            pl.BlockSpec((HS, C), lambda i: (0, 0)),
        ],
        out_specs=pl.BlockSpec((TB, C), lambda i: (i, 0)),
        out_shape=jax.ShapeDtypeStruct((N, C), jnp.float32),
        compiler_params=pltpu.CompilerParams(
            dimension_semantics=("arbitrary",)),
    )(xf, sg, su, sd)


def _ffn_body(be_ref, xs_ref, eg_ref, eu_ref, ed_ref, ys_ref):
    i = pl.program_id(0)

    @pl.when(be_ref[i] < E)
    def _():
        x = xs_ref[...]
        g = jnp.dot(x, eg_ref[0], preferred_element_type=jnp.float32)
        u = jnp.dot(x, eu_ref[0], preferred_element_type=jnp.float32)
        h = g * jax.nn.sigmoid(g) * u
        ys_ref[...] = jnp.dot(h, ed_ref[0], preferred_element_type=jnp.float32)


def _ffn(block_expert, xs, eg, eu, ed):
    def emap(i, be):
        return (jnp.minimum(be[i], E - 1), 0, 0)

    def edmap(i, be):
        return (jnp.minimum(be[i], E - 1), 0, 0)

    grid_spec = pltpu.PrefetchScalarGridSpec(
        num_scalar_prefetch=1,
        grid=(MAXB,),
        in_specs=[
            pl.BlockSpec((BLK, C), lambda i, be: (i, 0)),
            pl.BlockSpec((1, C, H), emap),
            pl.BlockSpec((1, C, H), emap),
            pl.BlockSpec((1, H, C), edmap),
        ],
        out_specs=pl.BlockSpec((BLK, C), lambda i, be: (i, 0)),
    )
    return pl.pallas_call(
        _ffn_body,
        grid_spec=grid_spec,
        out_shape=jax.ShapeDtypeStruct((PMAX, C), jnp.float32),
        compiler_params=pltpu.CompilerParams(
            dimension_semantics=("arbitrary",)),
    )(block_expert, xs, eg, eu, ed)


AW = NK // NW
HC = AW // 2


def _sc_scatter_dispatch(xf, sidx):
    mesh = plsc.VectorSubcoreMesh(core_axis_name="c", subcore_axis_name="s")

    @functools.partial(
        pl.kernel,
        mesh=mesh,
        out_type=jax.ShapeDtypeStruct((PMAX, C), jnp.float32),
        scratch_types=[
            pltpu.VMEM((2, HC), jnp.int32),
            pltpu.VMEM((HC, C), jnp.float32),
            pltpu.VMEM((HC, C), jnp.float32),
            pltpu.SemaphoreType.DMA,
            pltpu.SemaphoreType.DMA,
            pltpu.SemaphoreType.DMA,
            pltpu.SemaphoreType.DMA,
        ],
    )
    def k(xf_hbm, sidx_hbm, xs_hbm, idx_v, b0, b1, s0, s1, w0, w1):
        wid = lax.axis_index("s") * SC_CORES + lax.axis_index("c")
        base = (wid * AW) % N
        pltpu.sync_copy(sidx_hbm.at[wid], idx_v)
        r0 = pltpu.async_copy(xf_hbm.at[pl.ds(base, HC)], b0, s0)
        r1 = pltpu.async_copy(xf_hbm.at[pl.ds(base + HC, HC)], b1, s1)
        r0.wait()
        c0 = pltpu.async_copy(b0, xs_hbm.at[idx_v.at[0]], w0)
        r1.wait()
        c1 = pltpu.async_copy(b1, xs_hbm.at[idx_v.at[1]], w1)
        c0.wait()
        c1.wait()

    return k(xf, sidx)


def _sc_gather_out(ys, pos):
    mesh = plsc.VectorSubcoreMesh(core_axis_name="c", subcore_axis_name="s")

    @functools.partial(
        pl.kernel,
        mesh=mesh,
        out_type=jax.ShapeDtypeStruct((K * N, C), jnp.float32),
        scratch_types=[
            pltpu.VMEM((K, TW), jnp.int32),
            pltpu.VMEM((TW, C), jnp.float32),
            pltpu.VMEM((TW, C), jnp.float32),
            pltpu.SemaphoreType.DMA,
            pltpu.SemaphoreType.DMA,
            pltpu.SemaphoreType.DMA,
            pltpu.SemaphoreType.DMA,
        ],
    )
    def k(ys_hbm, pos_hbm, yg_hbm, idx_v, buf0, buf1, s0, s1, w0, w1):
        wid = lax.axis_index("s") * SC_CORES + lax.axis_index("c")
        pltpu.sync_copy(pos_hbm.at[wid], idx_v)
        g0 = pltpu.async_copy(ys_hbm.at[idx_v.at[0]], buf0, s0)
        g1 = pltpu.async_copy(ys_hbm.at[idx_v.at[1]], buf1, s1)
        g0.wait()
        c0 = pltpu.async_copy(buf0, yg_hbm.at[pl.ds(wid * TW, TW)], w0)
        g1.wait()
        c1 = pltpu.async_copy(buf1, yg_hbm.at[pl.ds(N + wid * TW, TW)], w1)
        c0.wait()
        c1.wait()

    return k(ys, pos)


def _final_body(sh_ref, y1_ref, y2_ref, w1_ref, w2_ref, out_ref):
    out_ref[...] = (sh_ref[...] + y1_ref[...] * w1_ref[...]
                    + y2_ref[...] * w2_ref[...])


def _final_add(shared, yg, w1, w2):
    TB = 512
    return pl.pallas_call(
        _final_body,
        grid=(N // TB,),
        in_specs=[
            pl.BlockSpec((TB, C), lambda i: (i, 0)),
            pl.BlockSpec((TB, C), lambda i: (i, 0)),
            pl.BlockSpec((TB, C), lambda i: (i + N // TB, 0)),
            pl.BlockSpec((TB, 1), lambda i: (i, 0)),
            pl.BlockSpec((TB, 1), lambda i: (i, 0)),
        ],
        out_specs=pl.BlockSpec((TB, C), lambda i: (i, 0)),
        out_shape=jax.ShapeDtypeStruct((N, C), jnp.float32),
        compiler_params=pltpu.CompilerParams(
            dimension_semantics=("arbitrary",)),
    )(shared, yg, yg, w1, w2)


def _dispatch_meta(idx_en):
    expert_flat = idx_en.reshape(-1)
    onehot = (expert_flat[:, None]
              == jnp.arange(E, dtype=expert_flat.dtype)[None, :]
              ).astype(jnp.int32)
    cum = jnp.cumsum(onehot, axis=0)
    counts = cum[-1]
    pcounts = ((counts + BLK - 1) // BLK) * BLK
    pend = jnp.cumsum(pcounts)
    pstarts = pend - pcounts
    ppos = jnp.sum(onehot * (cum + pstarts[None, :] - 1),
                   axis=1).astype(jnp.int32)
    block_expert = jnp.sum(
        pend[None, :] <= (jnp.arange(MAXB, dtype=jnp.int32) * BLK)[:, None],
        axis=1).astype(jnp.int32)
    sidx = ppos.reshape(NW, 2, HC)
    pos = ppos.reshape(K, NW, TW).transpose(1, 0, 2)
    return sidx, pos, block_expert


def kernel(x, router_w, eg, eu, ed, sg, su, sd):
    xf = x.reshape(N, C)
    idx_en, w_en, aux = _router(xf, router_w)
    sidx, pos, block_expert = _dispatch_meta(idx_en)
    xs = _sc_scatter_dispatch(xf, sidx)
    shared = _shared(xf, sg, su, sd)
    xs, shared = lax.optimization_barrier((xs, shared))
    ys = _ffn(block_expert, xs, eg, eu, ed)
    yg = _sc_gather_out(ys, pos)
    w1 = w_en[0].reshape(N, 1)
    w2 = w_en[1].reshape(N, 1)
    final = _final_add(shared, yg, w1, w2)
    return final.reshape(x.shape), aux[0, 0]


# speedup vs baseline: 1.0934x; 1.0217x over previous
import functools

import jax
import jax.numpy as jnp
from jax import lax
from jax.experimental import pallas as pl
from jax.experimental.pallas import tpu as pltpu
from jax.experimental.pallas import tpu_sc as plsc

N = 2048
C = 768
E = 8
K = 2
H = 2048
HS = 2048
AUXC = 0.01

BLK = 256
NK = N * K
MAXB = NK // BLK + E
PMAX = MAXB * BLK

SC_CORES = 2
SC_SUBCORES = 16
NW = SC_CORES * SC_SUBCORES
TW = N // NW


_CC = 256
_NCH = N // _CC


def _router_body(xf_ref, rw_ref, w_ref, aux_ref, ppos_ref, be_ref):
    x = xf_ref[...]
    rw = rw_ref[...]
    logits = lax.dot_general(rw, x, (((1,), (1,)), ((), ())),
                             preferred_element_type=jnp.float32)
    e_iota = lax.broadcasted_iota(jnp.int32, (E, N), 0)
    m1 = jnp.max(logits, axis=0, keepdims=True)
    i1 = jnp.min(jnp.where(logits == m1, e_iota, E), axis=0, keepdims=True)
    masked = jnp.where(e_iota == i1, -jnp.inf, logits)
    m2 = jnp.max(masked, axis=0, keepdims=True)
    i2 = jnp.min(jnp.where(masked == m2, e_iota, E), axis=0, keepdims=True)
    t = jnp.exp(m2 - m1)
    w1 = 1.0 / (1.0 + t)
    w2 = t / (1.0 + t)
    z = jnp.exp(logits - m1)
    probs = z / jnp.sum(z, axis=0, keepdims=True)
    mean_probs = jnp.sum(probs, axis=1, keepdims=True) / N
    sel1 = (e_iota == i1).astype(jnp.float32)
    sel2 = (e_iota == i2).astype(jnp.float32)
    frac = jnp.sum(sel1 + sel2, axis=1, keepdims=True) / N
    aux_ref[...] = (AUXC * jnp.sum(frac * mean_probs)).reshape(1, 1)
    w_ref[...] = jnp.concatenate([w1, w2], axis=0)

    c1 = jnp.sum(sel1, axis=1, keepdims=True)
    counts = c1 + jnp.sum(sel2, axis=1, keepdims=True)
    pc = jnp.ceil(counts / BLK) * BLK
    le = (lax.broadcasted_iota(jnp.int32, (E, E), 1)
          <= lax.broadcasted_iota(jnp.int32, (E, E), 0)).astype(jnp.float32)
    pend = jnp.dot(le, pc, preferred_element_type=jnp.float32)
    pstart = pend - pc
    tri = (lax.broadcasted_iota(jnp.int32, (_CC, _CC), 0)
           < lax.broadcasted_iota(jnp.int32, (_CC, _CC), 1)
           ).astype(jnp.float32)
    carry1 = jnp.zeros((E, 1), jnp.float32)
    carry2 = jnp.zeros((E, 1), jnp.float32)
    a1_parts = []
    a2_parts = []
    for c in range(_NCH):
        b1 = sel1[:, c * _CC:(c + 1) * _CC]
        b2 = sel2[:, c * _CC:(c + 1) * _CC]
        a1_parts.append(
            jnp.dot(b1, tri, preferred_element_type=jnp.float32) + carry1)
        a2_parts.append(
            jnp.dot(b2, tri, preferred_element_type=jnp.float32) + carry2)
        carry1 = carry1 + jnp.sum(b1, axis=1, keepdims=True)
        carry2 = carry2 + jnp.sum(b2, axis=1, keepdims=True)
    a1 = jnp.concatenate(a1_parts, axis=1)
    a2 = jnp.concatenate(a2_parts, axis=1)
    r1 = jnp.sum(sel1 * (a1 + pstart), axis=0, keepdims=True)
    r2 = jnp.sum(sel2 * (a2 + pstart + c1), axis=0, keepdims=True)
    ppos_ref[...] = jnp.concatenate([r1, r2], axis=0).astype(jnp.int32)
    rblk = (lax.broadcasted_iota(jnp.int32, (1, MAXB), 1) * BLK
            ).astype(jnp.float32)
    be_ref[...] = jnp.sum((pend <= rblk).astype(jnp.int32), axis=0,
                          keepdims=True)


def _router(xf, router_w):
    return pl.pallas_call(
        _router_body,
        out_shape=(
            jax.ShapeDtypeStruct((K, N), jnp.float32),
            jax.ShapeDtypeStruct((1, 1), jnp.float32),
            jax.ShapeDtypeStruct((K, N), jnp.int32),
            jax.ShapeDtypeStruct((1, MAXB), jnp.int32),
        ),
    )(xf, router_w)


def _shared_body(x_ref, sg_ref, su_ref, sd_ref, out_ref):
    x = x_ref[...]
    g = jnp.dot(x, sg_ref[...], preferred_element_type=jnp.float32)
    u = jnp.dot(x, su_ref[...], preferred_element_type=jnp.float32)
    h = g * jax.nn.sigmoid(g) * u
    out_ref[...] = jnp.dot(h, sd_ref[...], preferred_element_type=jnp.float32)


def _shared(xf, sg, su, sd):
    TB = 512
    return pl.pallas_call(
        _shared_body,
        grid=(N // TB,),
        in_specs=[
            pl.BlockSpec((TB, C), lambda i: (i, 0)),
            pl.BlockSpec((C, HS), lambda i: (0, 0)),
            pl.BlockSpec((C, HS), lambda i: (0, 0)),
            pl.BlockSpec((HS, C), lambda i: (0, 0)),
        ],
        out_specs=pl.BlockSpec((TB, C), lambda i: (i, 0)),
        out_shape=jax.ShapeDtypeStruct((N, C), jnp.float32),
        compiler_params=pltpu.CompilerParams(
            dimension_semantics=("arbitrary",)),
    )(xf, sg, su, sd)


def _ffn_body(be_ref, xs_ref, eg_ref, eu_ref, ed_ref, ys_ref):
    i = pl.program_id(0)

    @pl.when(be_ref[i] < E)
    def _():
        x = xs_ref[...]
        g = jnp.dot(x, eg_ref[0], preferred_element_type=jnp.float32)
        u = jnp.dot(x, eu_ref[0], preferred_element_type=jnp.float32)
        h = g * jax.nn.sigmoid(g) * u
        ys_ref[...] = jnp.dot(h, ed_ref[0], preferred_element_type=jnp.float32)


def _ffn(block_expert, xs, eg, eu, ed):
    def emap(i, be):
        return (jnp.minimum(be[i], E - 1), 0, 0)

    def edmap(i, be):
        return (jnp.minimum(be[i], E - 1), 0, 0)

    grid_spec = pltpu.PrefetchScalarGridSpec(
        num_scalar_prefetch=1,
        grid=(MAXB,),
        in_specs=[
            pl.BlockSpec((BLK, C), lambda i, be: (i, 0)),
            pl.BlockSpec((1, C, H), emap),
            pl.BlockSpec((1, C, H), emap),
            pl.BlockSpec((1, H, C), edmap),
        ],
        out_specs=pl.BlockSpec((BLK, C), lambda i, be: (i, 0)),
    )
    return pl.pallas_call(
        _ffn_body,
        grid_spec=grid_spec,
        out_shape=jax.ShapeDtypeStruct((PMAX, C), jnp.float32),
        compiler_params=pltpu.CompilerParams(
            dimension_semantics=("arbitrary",)),
    )(block_expert, xs, eg, eu, ed)


AW = NK // NW
HC = AW // 2


def _sc_scatter_dispatch(xf, sidx):
    mesh = plsc.VectorSubcoreMesh(core_axis_name="c", subcore_axis_name="s")

    @functools.partial(
        pl.kernel,
        mesh=mesh,
        out_type=jax.ShapeDtypeStruct((PMAX, C), jnp.float32),
        scratch_types=[
            pltpu.VMEM((2, HC), jnp.int32),
            pltpu.VMEM((HC, C), jnp.float32),
            pltpu.VMEM((HC, C), jnp.float32),
            pltpu.SemaphoreType.DMA,
            pltpu.SemaphoreType.DMA,
            pltpu.SemaphoreType.DMA,
            pltpu.SemaphoreType.DMA,
        ],
    )
    def k(xf_hbm, sidx_hbm, xs_hbm, idx_v, b0, b1, s0, s1, w0, w1):
        wid = lax.axis_index("s") * SC_CORES + lax.axis_index("c")
        base = (wid * AW) % N
        pltpu.sync_copy(sidx_hbm.at[wid], idx_v)
        r0 = pltpu.async_copy(xf_hbm.at[pl.ds(base, HC)], b0, s0)
        r1 = pltpu.async_copy(xf_hbm.at[pl.ds(base + HC, HC)], b1, s1)
        r0.wait()
        c0 = pltpu.async_copy(b0, xs_hbm.at[idx_v.at[0]], w0)
        r1.wait()
        c1 = pltpu.async_copy(b1, xs_hbm.at[idx_v.at[1]], w1)
        c0.wait()
        c1.wait()

    return k(xf, sidx)


def _sc_gather_out(ys, pos):
    mesh = plsc.VectorSubcoreMesh(core_axis_name="c", subcore_axis_name="s")

    @functools.partial(
        pl.kernel,
        mesh=mesh,
        out_type=jax.ShapeDtypeStruct((K * N, C), jnp.float32),
        scratch_types=[
            pltpu.VMEM((K, TW), jnp.int32),
            pltpu.VMEM((TW, C), jnp.float32),
            pltpu.VMEM((TW, C), jnp.float32),
            pltpu.SemaphoreType.DMA,
            pltpu.SemaphoreType.DMA,
            pltpu.SemaphoreType.DMA,
            pltpu.SemaphoreType.DMA,
        ],
    )
    def k(ys_hbm, pos_hbm, yg_hbm, idx_v, buf0, buf1, s0, s1, w0, w1):
        wid = lax.axis_index("s") * SC_CORES + lax.axis_index("c")
        pltpu.sync_copy(pos_hbm.at[wid], idx_v)
        g0 = pltpu.async_copy(ys_hbm.at[idx_v.at[0]], buf0, s0)
        g1 = pltpu.async_copy(ys_hbm.at[idx_v.at[1]], buf1, s1)
        g0.wait()
        c0 = pltpu.async_copy(buf0, yg_hbm.at[pl.ds(wid * TW, TW)], w0)
        g1.wait()
        c1 = pltpu.async_copy(buf1, yg_hbm.at[pl.ds(N + wid * TW, TW)], w1)
        c0.wait()
        c1.wait()

    return k(ys, pos)


def _final_body(sh_ref, y1_ref, y2_ref, w1_ref, w2_ref, out_ref):
    out_ref[...] = (sh_ref[...] + y1_ref[...] * w1_ref[...]
                    + y2_ref[...] * w2_ref[...])


def _final_add(shared, yg, w1, w2):
    TB = 512
    return pl.pallas_call(
        _final_body,
        grid=(N // TB,),
        in_specs=[
            pl.BlockSpec((TB, C), lambda i: (i, 0)),
            pl.BlockSpec((TB, C), lambda i: (i, 0)),
            pl.BlockSpec((TB, C), lambda i: (i + N // TB, 0)),
            pl.BlockSpec((TB, 1), lambda i: (i, 0)),
            pl.BlockSpec((TB, 1), lambda i: (i, 0)),
        ],
        out_specs=pl.BlockSpec((TB, C), lambda i: (i, 0)),
        out_shape=jax.ShapeDtypeStruct((N, C), jnp.float32),
        compiler_params=pltpu.CompilerParams(
            dimension_semantics=("arbitrary",)),
    )(shared, yg, yg, w1, w2)


def kernel(x, router_w, eg, eu, ed, sg, su, sd):
    xf = x.reshape(N, C)
    w_en, aux, ppos, be = _router(xf, router_w)
    sidx = ppos.reshape(NW, 2, HC)
    pos = ppos.reshape(K, NW, TW).transpose(1, 0, 2)
    block_expert = be.reshape(MAXB)
    xs = _sc_scatter_dispatch(xf, sidx)
    shared = _shared(xf, sg, su, sd)
    xs, shared = lax.optimization_barrier((xs, shared))
    ys = _ffn(block_expert, xs, eg, eu, ed)
    yg = _sc_gather_out(ys, pos)
    w1 = w_en[0].reshape(N, 1)
    w2 = w_en[1].reshape(N, 1)
    final = _final_add(shared, yg, w1, w2)
    return final.reshape(x.shape), aux[0, 0]
